# baseline (speedup 1.0000x reference)
import jax
import jax.numpy as jnp
from jax import lax
from jax.experimental import pallas as pl
from jax.experimental.pallas import tpu as pltpu

_N = 10000
_DIM = 128
_K = 8000
_IDX_BITS = 14

_TILES = 8
_TILE_N = 1280
_NP = _TILES * _TILE_N
_TPR = _TILE_N // 128

_DN_T = (((1,), (1,)), ((), ()))
_DN = (((1,), (0,)), ((), ()))


def _body(x_ref, WzT_ref, WhT_ref, bxz_ref, bhz_ref, bxh_ref, bhh_ref, pw_ref,
          l1W_ref, l1b_ref, l2W_ref, l2b_ref, out_ref,
          hT_scr, key_scr):
    i = pl.program_id(0)

    @pl.when(i < _TILES)
    def _dense():
        x = x_ref[...]
        az = (lax.dot_general(WzT_ref[...], x, _DN_T,
                              preferred_element_type=jnp.float32)
              + bxz_ref[...]) + bhz_ref[...]
        ah = (lax.dot_general(WhT_ref[...], x, _DN_T,
                              preferred_element_type=jnp.float32)
              + bxh_ref[...]) + bhh_ref[...]
        z = jax.nn.sigmoid(az)
        htil = jnp.tanh(ah)
        hT = jax.nn.relu((1.0 - z) * htil)

        pw = pw_ref[...]
        norm = jnp.sqrt(jnp.sum(pw * pw))
        sraw = lax.dot_general(pw, hT, _DN,
                               preferred_element_type=jnp.float32)
        sc = jnp.tanh(sraw / norm)

        bits = lax.bitcast_convert_type(sc, jnp.int32)
        key = jnp.where(bits >= 0, bits, bits ^ jnp.int32(0x7FFFFFFF))
        ubt = lax.bitcast_convert_type(key ^ jnp.int32(-2147483648),
                                       jnp.uint32)
        col = i * _TILE_N + lax.broadcasted_iota(jnp.int32, (1, _TILE_N), 1)
        valid = col < _N
        ubt = jnp.where(valid, ubt, jnp.uint32(0))
        hT_scr[:, pl.ds(i * _TILE_N, _TILE_N)] = jnp.where(valid, hT, 0.0)
        key_scr[:, pl.ds(i * _TILE_N, _TILE_N)] = ubt

    @pl.when(i == _TILES)
    def _finale():
        _select_reduce(hT_scr, key_scr, l1W_ref, l1b_ref,
                       l2W_ref, l2b_ref, out_ref)


def _select_reduce(hT_scr, key_scr, l1W_ref, l1b_ref,
                   l2W_ref, l2b_ref, out_ref):
    hT = hT_scr[...]
    ub = key_scr[...]
    ubp = jnp.reshape(ub, (80, 128))

    T = jnp.uint32(0)
    for hi in range(31, -1, -2):
        b1 = jnp.uint32(1 << hi)
        b2 = jnp.uint32(1 << (hi - 1))
        c_hi = jnp.sum((ubp >= (T | b1)).astype(jnp.int32))
        c_hi2 = jnp.sum((ubp >= (T | b1 | b2)).astype(jnp.int32))
        c_lo = jnp.sum((ubp >= (T | b2)).astype(jnp.int32))
        T = jnp.where(c_hi >= _K,
                      jnp.where(c_hi2 >= _K, T | b1 | b2, T | b1),
                      jnp.where(c_lo >= _K, T | b2, T))

    c_gt = jnp.sum((ubp > T).astype(jnp.int32))
    m = _K - c_gt
    tiep = ubp == T
    idxp = (lax.broadcasted_iota(jnp.int32, (80, 128), 0) * 128
            + lax.broadcasted_iota(jnp.int32, (80, 128), 1))
    C = jnp.int32(0)
    for hi in range(_IDX_BITS - 1, -1, -2):
        b1 = jnp.int32(1 << hi)
        b2 = jnp.int32(1 << (hi - 1))
        f_hi = jnp.sum((tiep & (idxp < (C | b1))).astype(jnp.int32))
        f_hi2 = jnp.sum((tiep & (idxp < (C | b1 | b2))).astype(jnp.int32))
        f_lo = jnp.sum((tiep & (idxp < (C | b2))).astype(jnp.int32))
        C = jnp.where(f_hi <= m,
                      jnp.where(f_hi2 <= m, C | b1 | b2, C | b1),
                      jnp.where(f_lo <= m, C | b2, C))

    tie = ub == T
    idx = lax.broadcasted_iota(jnp.int32, (1, _NP), 1)
    mask = (ub > T) | (tie & (idx < C))
    kx = lax.bitcast_convert_type(ub, jnp.int32) ^ jnp.int32(-2147483648)
    bits = jnp.where(kx >= 0, kx, kx ^ jnp.int32(0x7FFFFFFF))
    sc = lax.bitcast_convert_type(bits, jnp.float32)
    scm = jnp.where(mask, sc, 0.0)
    xpm = hT * scm
    gmax = jnp.max(jnp.where(mask, xpm, -jnp.inf), axis=1, keepdims=True)
    gsum = jnp.sum(xpm, axis=1, keepdims=True)
    gmean = gsum / jnp.float32(_K)

    g = jnp.transpose(jnp.concatenate([gmax, gmean], axis=0),
                      (1, 0))
    t1 = jax.nn.relu(lax.dot_general(g, l1W_ref[...], _DN,
                                     preferred_element_type=jnp.float32)
                     + l1b_ref[...])
    out_ref[...] = (jnp.sum(t1 * l2W_ref[...], axis=1, keepdims=True)
                    + l2b_ref[...])


def _run(x, WzT, WhT, bxz, bhz, bxh, bhh, pw, l1W, l1b, l2W, l2b,
         *, interpret=False):
    full = lambda *s: pl.BlockSpec(s, lambda i: tuple(0 for _ in s))
    return pl.pallas_call(
        _body,
        grid=(_TILES + 1,),
        in_specs=[
            pl.BlockSpec((_TILE_N, _DIM),
                         lambda i: (jnp.minimum(i, _TILES - 1), 0)),
            full(_DIM, _DIM), full(_DIM, _DIM),
            full(_DIM, 1), full(_DIM, 1), full(_DIM, 1), full(_DIM, 1),
            full(1, _DIM),
            full(2 * _DIM, _DIM), full(1, _DIM),
            full(1, _DIM), full(1, 1),
        ],
        out_specs=pl.BlockSpec((1, 1), lambda i: (0, 0)),
        scratch_shapes=[
            pltpu.VMEM((_DIM, _NP), jnp.float32),
            pltpu.VMEM((1, _NP), jnp.uint32),
        ],
        out_shape=jax.ShapeDtypeStruct((1, 1), jnp.float32),
        interpret=interpret,
    )(x, WzT, WhT, bxz, bhz, bxh, bhh, pw, l1W, l1b, l2W, l2b)


def kernel(x, edge_index, edge_attr, batch, W_xz, b_xz, W_hz, b_hz,
           W_xr, b_xr, W_hr, b_hr, W_xh, b_xh, W_hh, b_hh,
           pool_w, lin1_W, lin1_b, lin2_W, lin2_b):
    return _run(x, W_xz.T, W_xh.T,
                b_xz.reshape(_DIM, 1), b_hz.reshape(_DIM, 1),
                b_xh.reshape(_DIM, 1), b_hh.reshape(_DIM, 1),
                pool_w.reshape(1, _DIM),
                lin1_W, lin1_b.reshape(1, _DIM),
                lin2_W.reshape(1, _DIM), lin2_b.reshape(1, 1))


# speedup vs baseline: 1.1157x; 1.1157x over previous
import jax
import jax.numpy as jnp
from jax import lax
from jax.experimental import pallas as pl

_N = 10000
_DIM = 128
_K = 8000
_IDX_BITS = 14

_DN_T = (((1,), (1,)), ((), ()))
_DN = (((1,), (0,)), ((), ()))


def _body(x_ref, WzT_ref, WhT_ref, bxz_ref, bhz_ref, bxh_ref, bhh_ref, pw_ref,
          l1W_ref, l1b_ref, l2W_ref, l2b_ref, out_ref):
    x = x_ref[...]
    az = (lax.dot_general(WzT_ref[...], x, _DN_T,
                          preferred_element_type=jnp.float32)
          + bxz_ref[...]) + bhz_ref[...]
    ah = (lax.dot_general(WhT_ref[...], x, _DN_T,
                          preferred_element_type=jnp.float32)
          + bxh_ref[...]) + bhh_ref[...]
    z = jax.nn.sigmoid(az)
    htil = jnp.tanh(ah)
    hT = jax.nn.relu((1.0 - z) * htil)

    pw = pw_ref[...]
    norm = jnp.sqrt(jnp.sum(pw * pw))
    sraw = lax.dot_general(pw, hT, _DN,
                           preferred_element_type=jnp.float32)
    sc = jnp.tanh(sraw / norm)

    bits = lax.bitcast_convert_type(sc, jnp.int32)
    key = jnp.where(bits >= 0, bits, bits ^ jnp.int32(0x7FFFFFFF))
    ub = lax.bitcast_convert_type(key ^ jnp.int32(-2147483648), jnp.uint32)

    ubp = jnp.reshape(
        jnp.concatenate([ub, jnp.zeros((1, 240), jnp.uint32)], axis=1),
        (80, 128))

    T = jnp.uint32(0)
    for hi in range(31, -1, -2):
        b1 = jnp.uint32(1 << hi)
        b2 = jnp.uint32(1 << (hi - 1))
        c_hi = jnp.sum((ubp >= (T | b1)).astype(jnp.int32))
        c_hi2 = jnp.sum((ubp >= (T | b1 | b2)).astype(jnp.int32))
        c_lo = jnp.sum((ubp >= (T | b2)).astype(jnp.int32))
        T = jnp.where(c_hi >= _K,
                      jnp.where(c_hi2 >= _K, T | b1 | b2, T | b1),
                      jnp.where(c_lo >= _K, T | b2, T))

    c_gt = jnp.sum((ubp > T).astype(jnp.int32))
    m = _K - c_gt
    tiep = ubp == T
    idxp = (lax.broadcasted_iota(jnp.int32, (80, 128), 0) * 128
            + lax.broadcasted_iota(jnp.int32, (80, 128), 1))
    C = jnp.int32(0)
    for hi in range(_IDX_BITS - 1, -1, -2):
        b1 = jnp.int32(1 << hi)
        b2 = jnp.int32(1 << (hi - 1))
        f_hi = jnp.sum((tiep & (idxp < (C | b1))).astype(jnp.int32))
        f_hi2 = jnp.sum((tiep & (idxp < (C | b1 | b2))).astype(jnp.int32))
        f_lo = jnp.sum((tiep & (idxp < (C | b2))).astype(jnp.int32))
        C = jnp.where(f_hi <= m,
                      jnp.where(f_hi2 <= m, C | b1 | b2, C | b1),
                      jnp.where(f_lo <= m, C | b2, C))

    tie = ub == T
    idx = lax.broadcasted_iota(jnp.int32, (1, _N), 1)
    mask = (ub > T) | (tie & (idx < C))
    scm = jnp.where(mask, sc, 0.0)
    xpm = hT * scm
    gmax = jnp.max(jnp.where(mask, xpm, -jnp.inf), axis=1, keepdims=True)
    gsum = jnp.sum(xpm, axis=1, keepdims=True)
    gmean = gsum / jnp.float32(_K)

    g = jnp.transpose(jnp.concatenate([gmax, gmean], axis=0),
                      (1, 0))
    t1 = jax.nn.relu(lax.dot_general(g, l1W_ref[...], _DN,
                                     preferred_element_type=jnp.float32)
                     + l1b_ref[...])
    out_ref[...] = (jnp.sum(t1 * l2W_ref[...], axis=1, keepdims=True)
                    + l2b_ref[...])


def _run(x, WzT, WhT, bxz, bhz, bxh, bhh, pw, l1W, l1b, l2W, l2b,
         *, interpret=False):
    return pl.pallas_call(
        _body,
        out_shape=jax.ShapeDtypeStruct((1, 1), jnp.float32),
        interpret=interpret,
    )(x, WzT, WhT, bxz, bhz, bxh, bhh, pw, l1W, l1b, l2W, l2b)


def kernel(x, edge_index, edge_attr, batch, W_xz, b_xz, W_hz, b_hz,
           W_xr, b_xr, W_hr, b_hr, W_xh, b_xh, W_hh, b_hh,
           pool_w, lin1_W, lin1_b, lin2_W, lin2_b):
    return _run(x, W_xz.T, W_xh.T,
                b_xz.reshape(_DIM, 1), b_hz.reshape(_DIM, 1),
                b_xh.reshape(_DIM, 1), b_hh.reshape(_DIM, 1),
                pool_w.reshape(1, _DIM),
                lin1_W, lin1_b.reshape(1, _DIM),
                lin2_W.reshape(1, _DIM), lin2_b.reshape(1, 1))


# speedup vs baseline: 1.1580x; 1.0379x over previous
import jax
import jax.numpy as jnp
from jax import lax
from jax.experimental import pallas as pl

_N = 10000
_DIM = 128
_K = 8000
_IDX_BITS = 14

_DN_T = (((1,), (1,)), ((), ()))
_DN = (((1,), (0,)), ((), ()))


def _body(x_ref, WzT_ref, WhT_ref, bxz_ref, bhz_ref, bxh_ref, bhh_ref, pw_ref,
          l1W_ref, l1b_ref, l2W_ref, l2b_ref, out_ref):
    x = x_ref[...]
    az = (lax.dot_general(WzT_ref[...], x, _DN_T,
                          preferred_element_type=jnp.float32)
          + bxz_ref[...]) + bhz_ref[...]
    ah = (lax.dot_general(WhT_ref[...], x, _DN_T,
                          preferred_element_type=jnp.float32)
          + bxh_ref[...]) + bhh_ref[...]
    z = jax.nn.sigmoid(az)
    htil = jnp.tanh(ah)
    hT = jax.nn.relu((1.0 - z) * htil)

    pw = pw_ref[...]
    norm = jnp.sqrt(jnp.sum(pw * pw))
    sraw = lax.dot_general(pw, hT, _DN,
                           preferred_element_type=jnp.float32)
    sc = jnp.tanh(sraw / norm)

    bits = lax.bitcast_convert_type(sc, jnp.int32)
    key = jnp.where(bits >= 0, bits, bits ^ jnp.int32(0x7FFFFFFF))
    ub = lax.bitcast_convert_type(key ^ jnp.int32(-2147483648), jnp.uint32)

    ubp = jnp.reshape(
        jnp.concatenate([ub, jnp.zeros((1, 240), jnp.uint32)], axis=1),
        (80, 128))

    T = jnp.uint32(0)
    for lo in range(28, -4, -4):
        digit = jnp.int32(0)
        for a in range(1, 16):
            cnt = jnp.sum((ubp >= (T | jnp.uint32(a << lo))).astype(jnp.int32))
            digit = digit + (cnt >= _K).astype(jnp.int32)
        T = T | (digit.astype(jnp.uint32) << lo)

    c_gt = jnp.sum((ubp > T).astype(jnp.int32))
    m = _K - c_gt
    tiep = ubp == T
    idxp = (lax.broadcasted_iota(jnp.int32, (80, 128), 0) * 128
            + lax.broadcasted_iota(jnp.int32, (80, 128), 1))
    C = jnp.int32(0)
    for lo, g in ((12, 2), (8, 4), (4, 4), (0, 4)):
        digit = jnp.int32(0)
        for a in range(1, 1 << g):
            f = jnp.sum((tiep & (idxp < (C | jnp.int32(a << lo)))).astype(jnp.int32))
            digit = digit + (f <= m).astype(jnp.int32)
        C = C | (digit << lo)

    tie = ub == T
    idx = lax.broadcasted_iota(jnp.int32, (1, _N), 1)
    mask = (ub > T) | (tie & (idx < C))
    scm = jnp.where(mask, sc, 0.0)
    xpm = hT * scm
    gmax = jnp.max(jnp.where(mask, xpm, -jnp.inf), axis=1, keepdims=True)
    gsum = jnp.sum(xpm, axis=1, keepdims=True)
    gmean = gsum / jnp.float32(_K)

    g = jnp.transpose(jnp.concatenate([gmax, gmean], axis=0),
                      (1, 0))
    t1 = jax.nn.relu(lax.dot_general(g, l1W_ref[...], _DN,
                                     preferred_element_type=jnp.float32)
                     + l1b_ref[...])
    out_ref[...] = (jnp.sum(t1 * l2W_ref[...], axis=1, keepdims=True)
                    + l2b_ref[...])


def _run(x, WzT, WhT, bxz, bhz, bxh, bhh, pw, l1W, l1b, l2W, l2b,
         *, interpret=False):
    return pl.pallas_call(
        _body,
        out_shape=jax.ShapeDtypeStruct((1, 1), jnp.float32),
        interpret=interpret,
    )(x, WzT, WhT, bxz, bhz, bxh, bhh, pw, l1W, l1b, l2W, l2b)


def kernel(x, edge_index, edge_attr, batch, W_xz, b_xz, W_hz, b_hz,
           W_xr, b_xr, W_hr, b_hr, W_xh, b_xh, W_hh, b_hh,
           pool_w, lin1_W, lin1_b, lin2_W, lin2_b):
    return _run(x, W_xz.T, W_xh.T,
                b_xz.reshape(_DIM, 1), b_hz.reshape(_DIM, 1),
                b_xh.reshape(_DIM, 1), b_hh.reshape(_DIM, 1),
                pool_w.reshape(1, _DIM),
                lin1_W, lin1_b.reshape(1, _DIM),
                lin2_W.reshape(1, _DIM), lin2_b.reshape(1, 1))
